# Initial kernel scaffold; baseline (speedup 1.0000x reference)
import jax
import jax.numpy as jnp
from jax.experimental import pallas as pl


def kernel(inputs, interp_coe):
    raise NotImplementedError("write your pallas kernel here")


# speedup vs baseline: 4.2429x; 4.2429x over previous
import functools

import numpy as np
import jax
import jax.numpy as jnp
from jax import lax
from jax.experimental import pallas as pl
from jax.experimental.pallas import tpu as pltpu
from jax.experimental.pallas import tpu_sc as plsc

_N = 1048576
_MESH = 2000
_W = 2 * _MESH + 1
_NW = 32
_PPW = _N // _NW
_C = 2048
_NCHUNK = _PPW // _C
_L = 16

_DELTA = np.float32(1.0) / np.float32(_MESH)
_OFFS = [dx * _W + dy for dx in range(3) for dy in range(3)]


def _basis(l):
    b0 = (l - 1.0) * (l - 2.0) * 0.5
    b1 = l * (2.0 - l)
    b2 = l * (l - 1.0) * 0.5
    return b0, b1, b2


def _cell(t):
    ei = jnp.minimum(t.astype(jnp.int32), _MESH - 1)
    loc = (t - ei.astype(jnp.float32)) * 2.0
    return ei, loc


@jax.jit
def _sc_interp(x, y, table):
    mesh = plsc.VectorSubcoreMesh(core_axis_name="c", subcore_axis_name="s")

    @functools.partial(
        pl.kernel,
        mesh=mesh,
        out_type=jax.ShapeDtypeStruct((_N,), jnp.float32),
        scratch_types=(
            [pltpu.VMEM((_C,), jnp.float32),
             pltpu.VMEM((_C,), jnp.float32)]
            + [pltpu.VMEM((_C,), jnp.int32) for _ in range(9)]
            + [pltpu.VMEM((_C,), jnp.float32) for _ in range(9)]
            + [pltpu.VMEM((_C,), jnp.float32),
               pltpu.SemaphoreType.DMA]
        ),
    )
    def k(x_hbm, y_hbm, tab_hbm, out_hbm, xv, yv, *rest):
        idxv = rest[0:9]
        coev = rest[9:18]
        ov = rest[18]
        sem = rest[19]
        wid = lax.axis_index("s") * 2 + lax.axis_index("c")
        base = wid * _PPW

        def chunk_body(c, carry):
            off = base + c * _C
            pltpu.sync_copy(x_hbm.at[pl.ds(off, _C)], xv)
            pltpu.sync_copy(y_hbm.at[pl.ds(off, _C)], yv)

            def build(i, carry2):
                s = i * _L
                tx = xv[pl.ds(s, _L)] / _DELTA
                ty = yv[pl.ds(s, _L)] / _DELTA
                ex, _ = _cell(tx)
                ey, _ = _cell(ty)
                fb = (ex * 2) * _W + ey * 2
                for kk in range(9):
                    idxv[kk][pl.ds(s, _L)] = fb + _OFFS[kk]
                return carry2

            lax.fori_loop(0, _C // _L, build, 0, unroll=False)

            copies = [
                pltpu.async_copy(tab_hbm.at[idxv[kk]], coev[kk], sem)
                for kk in range(9)
            ]
            for cp in copies:
                cp.wait()

            def accum(i, carry2):
                s = i * _L
                tx = xv[pl.ds(s, _L)] / _DELTA
                ty = yv[pl.ds(s, _L)] / _DELTA
                _, lx = _cell(tx)
                _, ly = _cell(ty)
                bx = _basis(lx)
                by = _basis(ly)
                acc = None
                for dx in range(3):
                    r = (coev[3 * dx + 0][pl.ds(s, _L)] * by[0]
                         + coev[3 * dx + 1][pl.ds(s, _L)] * by[1]
                         + coev[3 * dx + 2][pl.ds(s, _L)] * by[2])
                    acc = r * bx[dx] if acc is None else acc + r * bx[dx]
                ov[pl.ds(s, _L)] = acc
                return carry2

            lax.fori_loop(0, _C // _L, accum, 0, unroll=False)
            pltpu.sync_copy(ov, out_hbm.at[pl.ds(off, _C)])
            return carry

        lax.fori_loop(0, _NCHUNK, chunk_body, 0, unroll=False)

    return k(x, y, table)


def kernel(inputs, interp_coe):
    x = inputs[:, 0]
    y = inputs[:, 1]
    table = interp_coe.reshape(-1)
    return _sc_interp(x, y, table)


# speedup vs baseline: 4.7189x; 1.1122x over previous
import functools

import numpy as np
import jax
import jax.numpy as jnp
from jax import lax
from jax.experimental import pallas as pl
from jax.experimental.pallas import tpu as pltpu
from jax.experimental.pallas import tpu_sc as plsc

_N = 1048576
_MESH = 2000
_W = 2 * _MESH + 1
_NW = 32
_PPW = _N // _NW
_C = 2048
_NCHUNK = _PPW // _C
_L = 16

_DELTA = np.float32(1.0) / np.float32(_MESH)
_OFFS = [dx * _W + dy for dx in range(3) for dy in range(3)]


def _basis(l):
    b0 = (l - 1.0) * (l - 2.0) * 0.5
    b1 = l * (2.0 - l)
    b2 = l * (l - 1.0) * 0.5
    return b0, b1, b2


def _cell(t):
    ei = jnp.minimum(t.astype(jnp.int32), _MESH - 1)
    loc = (t - ei.astype(jnp.float32)) * 2.0
    return ei, loc


def _one_set():
    return ([pltpu.VMEM((_C,), jnp.float32),
             pltpu.VMEM((_C,), jnp.float32)]
            + [pltpu.VMEM((_C,), jnp.int32) for _ in range(9)]
            + [pltpu.VMEM((_C,), jnp.float32) for _ in range(9)]
            + [pltpu.VMEM((_C,), jnp.float32),
               pltpu.SemaphoreType.DMA])


@jax.jit
def _sc_interp(x, y, table):
    mesh = plsc.VectorSubcoreMesh(core_axis_name="c", subcore_axis_name="s")

    @functools.partial(
        pl.kernel,
        mesh=mesh,
        out_type=jax.ShapeDtypeStruct((_N,), jnp.float32),
        scratch_types=_one_set() + _one_set(),
    )
    def k(x_hbm, y_hbm, tab_hbm, out_hbm, *rest):
        sets = []
        for b in range(2):
            r = rest[b * 22:(b + 1) * 22]
            sets.append(dict(xv=r[0], yv=r[1], idxv=r[2:11], coev=r[11:20],
                             ov=r[20], sem=r[21]))
        wid = lax.axis_index("s") * 2 + lax.axis_index("c")
        base = wid * _PPW

        def load_and_fire(c, S):
            off = base + c * _C
            pltpu.sync_copy(x_hbm.at[pl.ds(off, _C)], S["xv"])
            pltpu.sync_copy(y_hbm.at[pl.ds(off, _C)], S["yv"])

            def build(i, carry2):
                s = i * _L
                ex, _ = _cell(S["xv"][pl.ds(s, _L)] / _DELTA)
                ey, _ = _cell(S["yv"][pl.ds(s, _L)] / _DELTA)
                fb = (ex * 2) * _W + ey * 2
                for kk in range(9):
                    S["idxv"][kk][pl.ds(s, _L)] = fb + _OFFS[kk]
                return carry2

            lax.fori_loop(0, _C // _L, build, 0, unroll=False)
            return [
                pltpu.async_copy(tab_hbm.at[S["idxv"][kk]], S["coev"][kk],
                                 S["sem"])
                for kk in range(9)
            ]

        def drain_and_accum(c, S, copies):
            for cp in copies:
                cp.wait()

            def accum(i, carry2):
                s = i * _L
                _, lx = _cell(S["xv"][pl.ds(s, _L)] / _DELTA)
                _, ly = _cell(S["yv"][pl.ds(s, _L)] / _DELTA)
                bx = _basis(lx)
                by = _basis(ly)
                acc = None
                for dx in range(3):
                    r = (S["coev"][3 * dx + 0][pl.ds(s, _L)] * by[0]
                         + S["coev"][3 * dx + 1][pl.ds(s, _L)] * by[1]
                         + S["coev"][3 * dx + 2][pl.ds(s, _L)] * by[2])
                    acc = r * bx[dx] if acc is None else acc + r * bx[dx]
                S["ov"][pl.ds(s, _L)] = acc
                return carry2

            lax.fori_loop(0, _C // _L, accum, 0, unroll=False)
            off = base + c * _C
            pltpu.sync_copy(S["ov"], out_hbm.at[pl.ds(off, _C)])

        pending = load_and_fire(0, sets[0])
        for c in range(1, _NCHUNK):
            nxt = load_and_fire(c, sets[c % 2])
            drain_and_accum(c - 1, sets[(c - 1) % 2], pending)
            pending = nxt
        drain_and_accum(_NCHUNK - 1, sets[(_NCHUNK - 1) % 2], pending)

    return k(x, y, table)


def kernel(inputs, interp_coe):
    x = inputs[:, 0]
    y = inputs[:, 1]
    table = interp_coe.reshape(-1)
    return _sc_interp(x, y, table)
